# baseline (speedup 1.0000x reference)
import functools

import jax
import jax.numpy as jnp
from jax import lax
from jax.experimental import pallas as pl
from jax.experimental.pallas import tpu as pltpu
from jax.experimental.pallas import tpu_sc as plsc

N = 10000
E = 320000
D_IN = 128
D_HID = 128
D_OUT = 64

NC = 2
NS = 16
NW = NC * NS
NPAD = 10240
K = 128
CB = 20
NCHT = 8
NCH0 = 6
NCH1 = NCHT - NCH0
NCHD = NCHT // NC
EPAD = NS * NCHT * CB * K
RPT = NPAD // NS
WR = 128
WCH = RPT // WR


def _sc_mesh():
    return plsc.VectorSubcoreMesh(
        core_axis_name="c", subcore_axis_name="s",
        num_cores=NC, num_subcores=NS)


@functools.partial(
    pl.kernel,
    out_type=jax.ShapeDtypeStruct((NC, NPAD), jnp.float32),
    mesh=_sc_mesh(),
    scratch_types=[
        pltpu.VMEM((NCHD, CB, K), jnp.int32),
        pltpu.VMEM((K,), jnp.float32),
        pltpu.VMEM((RPT,), jnp.float32),
        pltpu.VMEM_SHARED((NPAD,), jnp.float32),
    ],
)
def _deg(dst_hbm, out_hbm, dst_v, ones_v, bounce_v, deg_sh):
    c = lax.axis_index("c")
    s = lax.axis_index("s")
    pltpu.sync_copy(dst_hbm.at[s, pl.ds(c * NCHD, NCHD)], dst_v)
    zv = jnp.zeros((16,), jnp.float32)
    ov = jnp.ones((16,), jnp.float32)

    @pl.loop(0, RPT // 16, unroll=4)
    def _(i):
        bounce_v[pl.ds(i * 16, 16)] = zv

    for i in range(K // 16):
        ones_v[pl.ds(i * 16, 16)] = ov

    pltpu.sync_copy(bounce_v, deg_sh.at[pl.ds(s * RPT, RPT)])
    plsc.subcore_barrier()

    @pl.loop(0, NCHD)
    def _(ci):
        @pl.loop(0, CB)
        def _(b):
            pltpu.sync_copy(ones_v, deg_sh.at[dst_v.at[ci, b]], add=True)

    plsc.subcore_barrier()
    pltpu.sync_copy(deg_sh.at[pl.ds(s * RPT, RPT)], bounce_v)
    pltpu.sync_copy(bounce_v, out_hbm.at[c, pl.ds(s * RPT, RPT)])


def _make_agg(D):
    params = (None if D % 128 == 0
              else pltpu.CompilerParams(use_tc_tiling_on_sc=False))

    @functools.partial(
        pl.kernel,
        out_type=jax.ShapeDtypeStruct((NC, NPAD, D), jnp.float32),
        mesh=_sc_mesh(),
        compiler_params=params,
        scratch_types=[
            pltpu.VMEM((2, CB, K), jnp.int32),
            pltpu.VMEM((2, CB, K), jnp.int32),
            pltpu.VMEM((K, D), jnp.float32),
            pltpu.VMEM((K, D), jnp.float32),
            pltpu.VMEM_SHARED((NPAD, D), jnp.float32),
            pltpu.SemaphoreType.DMA,
            pltpu.SemaphoreType.DMA,
            pltpu.SemaphoreType.DMA,
        ],
    )
    def agg(g_hbm, src_hbm, dst_hbm, out_hbm, src_c, dst_c, rows0, rows1,
            acc_sh, semg0, semg1, semi):
        c = lax.axis_index("c")
        s = lax.axis_index("s")

        zv = jnp.zeros((16,), jnp.float32)
        rows = (rows0, rows1)
        semg = (semg0, semg1)

        @pl.loop(0, K, unroll=4)
        def _(r):
            for i in range(D // 16):
                rows0[r, pl.ds(i * 16, 16)] = zv

        @pl.loop(0, WCH)
        def _(i):
            pltpu.sync_copy(rows0.at[pl.ds(0, WR)],
                            acc_sh.at[pl.ds(s * RPT + i * WR, WR)])

        plsc.subcore_barrier()

        def run_chunks(lo, nch):
            pltpu.sync_copy(src_hbm.at[s, lo], src_c.at[0])
            pltpu.sync_copy(dst_hbm.at[s, lo], dst_c.at[0])
            pltpu.async_copy(g_hbm.at[src_c.at[0, 0]], rows0, semg0)
            pltpu.async_copy(g_hbm.at[src_c.at[0, 1]], rows1, semg1)
            for k in range(nch):
                ci = lo + k
                p = k % 2
                if k + 1 < nch:
                    pltpu.async_copy(
                        src_hbm.at[s, ci + 1], src_c.at[1 - p], semi)
                    pltpu.async_copy(
                        dst_hbm.at[s, ci + 1], dst_c.at[1 - p], semi)

                @pl.loop(0, CB - 2, step=2)
                def _(b):
                    for t in range(2):
                        bb = b + t
                        pltpu.make_async_copy(
                            g_hbm.at[src_c.at[p, bb]], rows[t],
                            semg[t]).wait()
                        pltpu.sync_copy(
                            rows[t], acc_sh.at[dst_c.at[p, bb]], add=True)
                        pltpu.async_copy(
                            g_hbm.at[src_c.at[p, bb + 2]], rows[t], semg[t])

                if k + 1 < nch:
                    pltpu.make_async_copy(
                        src_hbm.at[s, ci + 1], src_c.at[1 - p], semi).wait()
                    pltpu.make_async_copy(
                        dst_hbm.at[s, ci + 1], dst_c.at[1 - p], semi).wait()
                for t, bb in ((0, CB - 2), (1, CB - 1)):
                    pltpu.make_async_copy(
                        g_hbm.at[src_c.at[p, bb]], rows[t], semg[t]).wait()
                    pltpu.sync_copy(
                        rows[t], acc_sh.at[dst_c.at[p, bb]], add=True)
                    if k + 1 < nch:
                        pltpu.async_copy(
                            g_hbm.at[src_c.at[1 - p, t]], rows[t], semg[t])

        @pl.when(c == 0)
        def _():
            run_chunks(0, NCH0)

        @pl.when(c == 1)
        def _():
            run_chunks(NCH0, NCH1)

        plsc.subcore_barrier()

        for i in range(WCH):
            t = i % 2
            if i >= 2:
                off_p = s * RPT + (i - 2) * WR
                pltpu.make_async_copy(
                    rows[t], out_hbm.at[c, pl.ds(off_p, WR)], semg[t]).wait()
            off = s * RPT + i * WR
            pltpu.sync_copy(acc_sh.at[pl.ds(off, WR)], rows[t])
            pltpu.async_copy(rows[t], out_hbm.at[c, pl.ds(off, WR)], semg[t])

        for i in range(max(WCH - 2, 0), WCH):
            t = i % 2
            off = s * RPT + i * WR
            pltpu.make_async_copy(
                rows[t], out_hbm.at[c, pl.ds(off, WR)], semg[t]).wait()

    return agg


_agg128 = _make_agg(D_HID)
_agg64 = _make_agg(D_OUT)


def _lin1_body(deg2_ref, x_ref, w1t_ref, dis_ref, g1_ref):
    deg = deg2_ref[:, 0:1] + deg2_ref[:, 1:2] + 1.0
    dis = lax.rsqrt(deg)
    dis_ref[...] = dis
    h = jnp.dot(x_ref[...], w1t_ref[...],
                preferred_element_type=jnp.float32,
                precision=lax.Precision.HIGHEST)
    g1_ref[...] = h * dis


def _mid_body(a0_ref, a1_ref, g1_ref, dis_ref, b1_ref, w2t_ref, g2_ref):
    dis = dis_ref[...]
    acc = a0_ref[...] + a1_ref[...] + g1_ref[...]
    h = jnp.maximum(acc * dis + b1_ref[...], 0.0)
    g2_ref[...] = jnp.dot(h, w2t_ref[...],
                          preferred_element_type=jnp.float32,
                          precision=lax.Precision.HIGHEST) * dis


def _out_body(a0_ref, a1_ref, g2_ref, dis_ref, b2_ref, o_ref):
    o_ref[...] = ((a0_ref[...] + a1_ref[...] + g2_ref[...]) * dis_ref[...]
                  + b2_ref[...])


def _f32(shape):
    return jax.ShapeDtypeStruct(shape, jnp.float32)


def kernel(x, edge_index, W1, b1, W2, b2):
    ei = edge_index.astype(jnp.int32)
    pad_idx = jnp.full((EPAD - E,), N, jnp.int32)
    srcp = jnp.concatenate([ei[0], pad_idx]).reshape(NS, NCHT, CB, K)
    dstp = jnp.concatenate([ei[1], pad_idx]).reshape(NS, NCHT, CB, K)
    xp = jnp.pad(x, ((0, NPAD - N), (0, 0)))

    deg_parts = _deg(dstp)
    deg2 = deg_parts.T

    dis, g1 = pl.pallas_call(
        _lin1_body,
        out_shape=[_f32((NPAD, 1)), _f32((NPAD, D_HID))],
    )(deg2, xp, W1.T)

    acc1 = _agg128(g1, srcp, dstp)

    g2 = pl.pallas_call(
        _mid_body,
        out_shape=_f32((NPAD, D_OUT)),
    )(acc1[0], acc1[1], g1, dis, b1.reshape(1, D_HID), W2.T)

    acc2 = _agg64(g2, srcp, dstp)

    out_full = pl.pallas_call(
        _out_body,
        out_shape=_f32((NPAD, D_OUT)),
    )(acc2[0], acc2[1], g2, dis, b2.reshape(1, D_OUT))

    return out_full[:N]


# speedup vs baseline: 1.0006x; 1.0006x over previous
import functools

import jax
import jax.numpy as jnp
from jax import lax
from jax.experimental import pallas as pl
from jax.experimental.pallas import tpu as pltpu
from jax.experimental.pallas import tpu_sc as plsc

N = 10000
E = 320000
D_IN = 128
D_HID = 128
D_OUT = 64

NC = 2
NS = 16
NW = NC * NS
NPAD = 10240
K = 128
CB = 20
NCHT = 8
NCH0 = 6
NCH1 = NCHT - NCH0
NCHD = NCHT // NC
EPAD = NS * NCHT * CB * K
RPT = NPAD // NS
WR = 128
WCH = RPT // WR


def _sc_mesh():
    return plsc.VectorSubcoreMesh(
        core_axis_name="c", subcore_axis_name="s",
        num_cores=NC, num_subcores=NS)


@functools.partial(
    pl.kernel,
    out_type=jax.ShapeDtypeStruct((NC, NPAD), jnp.float32),
    mesh=_sc_mesh(),
    scratch_types=[
        pltpu.VMEM((NCHD, CB, K), jnp.int32),
        pltpu.VMEM((K,), jnp.float32),
        pltpu.VMEM((RPT,), jnp.float32),
        pltpu.VMEM_SHARED((NPAD,), jnp.float32),
    ],
)
def _deg(dst_hbm, out_hbm, dst_v, ones_v, bounce_v, deg_sh):
    c = lax.axis_index("c")
    s = lax.axis_index("s")
    pltpu.sync_copy(dst_hbm.at[s, pl.ds(c * NCHD, NCHD)], dst_v)
    zv = jnp.zeros((16,), jnp.float32)
    ov = jnp.ones((16,), jnp.float32)

    @pl.loop(0, RPT // 16, unroll=4)
    def _(i):
        bounce_v[pl.ds(i * 16, 16)] = zv

    for i in range(K // 16):
        ones_v[pl.ds(i * 16, 16)] = ov

    pltpu.sync_copy(bounce_v, deg_sh.at[pl.ds(s * RPT, RPT)])
    plsc.subcore_barrier()

    @pl.loop(0, NCHD)
    def _(ci):
        @pl.loop(0, CB)
        def _(b):
            pltpu.sync_copy(ones_v, deg_sh.at[dst_v.at[ci, b]], add=True)

    plsc.subcore_barrier()
    pltpu.sync_copy(deg_sh.at[pl.ds(s * RPT, RPT)], bounce_v)
    pltpu.sync_copy(bounce_v, out_hbm.at[c, pl.ds(s * RPT, RPT)])


def _make_agg(D):
    params = (None if D % 128 == 0
              else pltpu.CompilerParams(use_tc_tiling_on_sc=False))

    @functools.partial(
        pl.kernel,
        out_type=jax.ShapeDtypeStruct((NC, NPAD, D), jnp.float32),
        mesh=_sc_mesh(),
        compiler_params=params,
        scratch_types=[
            pltpu.VMEM((2, CB, K), jnp.int32),
            pltpu.VMEM((2, CB, K), jnp.int32),
            pltpu.VMEM((K, D), jnp.float32),
            pltpu.VMEM((K, D), jnp.float32),
            pltpu.VMEM_SHARED((NPAD, D), jnp.float32),
            pltpu.SemaphoreType.DMA,
            pltpu.SemaphoreType.DMA,
            pltpu.SemaphoreType.DMA,
        ],
    )
    def agg(g_hbm, src_hbm, dst_hbm, out_hbm, src_c, dst_c, rows0, rows1,
            acc_sh, semg0, semg1, semi):
        c = lax.axis_index("c")
        s = lax.axis_index("s")

        zv = jnp.zeros((16,), jnp.float32)
        rows = (rows0, rows1)
        semg = (semg0, semg1)

        @pl.loop(0, K, unroll=4)
        def _(r):
            for i in range(D // 16):
                rows0[r, pl.ds(i * 16, 16)] = zv

        @pl.loop(0, WCH)
        def _(i):
            pltpu.sync_copy(rows0.at[pl.ds(0, WR)],
                            acc_sh.at[pl.ds(s * RPT + i * WR, WR)])

        plsc.subcore_barrier()

        def run_chunks(lo, nch):
            pltpu.sync_copy(src_hbm.at[s, lo], src_c.at[0])
            pltpu.sync_copy(dst_hbm.at[s, lo], dst_c.at[0])
            for k in range(nch):
                ci = lo + k
                p = k % 2
                if k + 1 < nch:
                    pltpu.async_copy(
                        src_hbm.at[s, ci + 1], src_c.at[1 - p], semi)
                    pltpu.async_copy(
                        dst_hbm.at[s, ci + 1], dst_c.at[1 - p], semi)
                pltpu.async_copy(g_hbm.at[src_c.at[p, 0]], rows0, semg0)
                pltpu.async_copy(g_hbm.at[src_c.at[p, 1]], rows1, semg1)

                @pl.loop(0, CB, step=2)
                def _(b):
                    for t in range(2):
                        bb = b + t
                        pltpu.make_async_copy(
                            g_hbm.at[src_c.at[p, bb]], rows[t],
                            semg[t]).wait()
                        pltpu.sync_copy(
                            rows[t], acc_sh.at[dst_c.at[p, bb]], add=True)

                        @pl.when(bb + 2 < CB)
                        def _():
                            pltpu.async_copy(
                                g_hbm.at[src_c.at[p, bb + 2]],
                                rows[t], semg[t])

                if k + 1 < nch:
                    pltpu.make_async_copy(
                        src_hbm.at[s, ci + 1], src_c.at[1 - p], semi).wait()
                    pltpu.make_async_copy(
                        dst_hbm.at[s, ci + 1], dst_c.at[1 - p], semi).wait()

        @pl.when(c == 0)
        def _():
            run_chunks(0, NCH0)

        @pl.when(c == 1)
        def _():
            run_chunks(NCH0, NCH1)

        plsc.subcore_barrier()

        for i in range(WCH):
            t = i % 2
            if i >= 2:
                off_p = s * RPT + (i - 2) * WR
                pltpu.make_async_copy(
                    rows[t], out_hbm.at[c, pl.ds(off_p, WR)], semg[t]).wait()
            off = s * RPT + i * WR
            pltpu.sync_copy(acc_sh.at[pl.ds(off, WR)], rows[t])
            pltpu.async_copy(rows[t], out_hbm.at[c, pl.ds(off, WR)], semg[t])

        for i in range(max(WCH - 2, 0), WCH):
            t = i % 2
            off = s * RPT + i * WR
            pltpu.make_async_copy(
                rows[t], out_hbm.at[c, pl.ds(off, WR)], semg[t]).wait()

    return agg


_agg128 = _make_agg(D_HID)
_agg64 = _make_agg(D_OUT)


def _lin1_body(deg2_ref, x_ref, w1t_ref, dis_ref, g1_ref):
    deg = deg2_ref[:, 0:1] + deg2_ref[:, 1:2] + 1.0
    dis = lax.rsqrt(deg)
    dis_ref[...] = dis
    h = jnp.dot(x_ref[...], w1t_ref[...],
                preferred_element_type=jnp.float32,
                precision=lax.Precision.HIGHEST)
    g1_ref[...] = h * dis


def _mid_body(a0_ref, a1_ref, g1_ref, dis_ref, b1_ref, w2t_ref, g2_ref):
    dis = dis_ref[...]
    acc = a0_ref[...] + a1_ref[...] + g1_ref[...]
    h = jnp.maximum(acc * dis + b1_ref[...], 0.0)
    g2_ref[...] = jnp.dot(h, w2t_ref[...],
                          preferred_element_type=jnp.float32,
                          precision=lax.Precision.HIGHEST) * dis


def _out_body(a0_ref, a1_ref, g2_ref, dis_ref, b2_ref, o_ref):
    o_ref[...] = ((a0_ref[...] + a1_ref[...] + g2_ref[...]) * dis_ref[...]
                  + b2_ref[...])


def _f32(shape):
    return jax.ShapeDtypeStruct(shape, jnp.float32)


def kernel(x, edge_index, W1, b1, W2, b2):
    ei = edge_index.astype(jnp.int32)
    pad_idx = jnp.full((EPAD - E,), N, jnp.int32)
    srcp = jnp.concatenate([ei[0], pad_idx]).reshape(NS, NCHT, CB, K)
    dstp = jnp.concatenate([ei[1], pad_idx]).reshape(NS, NCHT, CB, K)
    xp = jnp.pad(x, ((0, NPAD - N), (0, 0)))

    deg_parts = _deg(dstp)
    deg2 = deg_parts.T

    dis, g1 = pl.pallas_call(
        _lin1_body,
        out_shape=[_f32((NPAD, 1)), _f32((NPAD, D_HID))],
    )(deg2, xp, W1.T)

    acc1 = _agg128(g1, srcp, dstp)

    g2 = pl.pallas_call(
        _mid_body,
        out_shape=_f32((NPAD, D_OUT)),
    )(acc1[0], acc1[1], g1, dis, b1.reshape(1, D_HID), W2.T)

    acc2 = _agg64(g2, srcp, dstp)

    out_full = pl.pallas_call(
        _out_body,
        out_shape=_f32((NPAD, D_OUT)),
    )(acc2[0], acc2[1], g2, dis, b2.reshape(1, D_OUT))

    return out_full[:N]
